# baseline (speedup 1.0000x reference)
import functools

import jax
import jax.numpy as jnp
from jax import lax
from jax.experimental import pallas as pl
from jax.experimental.pallas import tpu as pltpu
from jax.experimental.pallas import tpu_sc as plsc

HIDDEN = 768
N_TOK = 8192
NC, NS, L = 2, 16, 16
NW = NC * NS
TOK_W = N_TOK // NW
TMAX = 32
HB = HIDDEN // L
U = 16
NWB = 3
NPB = 2

_SIZES = [8, 8, 16] + [32] * 7
assert sum(_SIZES) == TOK_W
_STARTS = [sum(_SIZES[:i]) for i in range(len(_SIZES))]
CHUNKS = list(zip(_STARTS, _SIZES))

_mesh = plsc.VectorSubcoreMesh(core_axis_name="c", subcore_axis_name="s")


@functools.partial(
    pl.kernel,
    mesh=_mesh,
    compiler_params=pltpu.CompilerParams(needs_layout_passes=False),
    out_type=jax.ShapeDtypeStruct((N_TOK, HIDDEN), jnp.float32),
    scratch_types=[
        pltpu.VMEM((TOK_W,), jnp.int32),
        pltpu.VMEM((TOK_W,), jnp.int32),
        pltpu.VMEM((TOK_W,), jnp.int32),
        pltpu.VMEM((NWB, TMAX, HIDDEN), jnp.float32),
        pltpu.VMEM((NPB, TMAX, HIDDEN), jnp.float32),
        pltpu.VMEM((2, HIDDEN), jnp.float32),
        pltpu.VMEM((TMAX, L), jnp.int32),
        pltpu.SemaphoreType.DMA,
        pltpu.SemaphoreType.DMA,
        pltpu.SemaphoreType.DMA,
        pltpu.SemaphoreType.DMA,
        pltpu.SemaphoreType.DMA,
        pltpu.SemaphoreType.DMA,
        pltpu.SemaphoreType.DMA,
        pltpu.SemaphoreType.DMA,
    ],
)
def _emb_kernel(idw_hbm, idp_hbm, idt_hbm, wtab_hbm, ptab_hbm, ttab_hbm,
                out_hbm, idw_v, idp_v, idt_v, wbuf, pbuf, tbuf, tidb_v,
                semw0, semw1, semw2, semp0, semp1, semo0, semo1, semo2):
    semw = (semw0, semw1, semw2)
    semp = (semp0, semp1)
    semo = (semo0, semo1, semo2)
    wid = lax.axis_index("s") * NC + lax.axis_index("c")
    base = wid * TOK_W
    pltpu.sync_copy(idw_hbm.at[pl.ds(base, TOK_W)], idw_v)
    pltpu.sync_copy(idp_hbm.at[pl.ds(base, TOK_W)], idp_v)
    pltpu.sync_copy(idt_hbm.at[pl.ds(base, TOK_W)], idt_v)
    pltpu.sync_copy(ttab_hbm, tbuf)

    def gathers(c):
        st, sz = CHUNKS[c]
        ws, ps = c % NWB, c % NPB
        gw = pltpu.async_copy(wtab_hbm.at[idw_v.at[pl.ds(st, sz)]],
                              wbuf.at[ws].at[pl.ds(0, sz)], semw[ws])
        gp = pltpu.async_copy(ptab_hbm.at[idp_v.at[pl.ds(st, sz)]],
                              pbuf.at[ps].at[pl.ds(0, sz)], semp[ps])
        return gw, gp

    pend_g = {0: gathers(0)}
    pend_o = {}
    for c in range(len(CHUNKS)):
        st, sz = CHUNKS[c]
        ws = c % NWB
        if c + 1 < len(CHUNKS):
            if c - 2 >= 0:
                pend_o.pop(c - 2).wait()
            pend_g[c + 1] = gathers(c + 1)
        @plsc.parallel_loop(0, sz, 1, unroll=4)
        def pre_body(t):
            tidb_v[t] = plsc.load_gather(
                idt_v, [jnp.full((L,), st + t, jnp.int32)])

        gw, gp = pend_g.pop(c)
        gw.wait()
        gp.wait()

        wv = wbuf.at[ws]
        pv = pbuf.at[c % NPB]

        def h_body(h, _):
            h16 = h * L
            a0 = tbuf[0, pl.ds(h16, L)]
            a1 = tbuf[1, pl.ds(h16, L)]

            @plsc.parallel_loop(0, sz, 1, unroll=min(U, sz))
            def t_loop(t):
                w = wv[t, pl.ds(h16, L)]
                p = pv[t, pl.ds(h16, L)]
                ti = tidb_v[t]
                wv[t, pl.ds(h16, L)] = w + p + jnp.where(ti != 0, a1, a0)

            return 0

        lax.fori_loop(0, HB, h_body, 0)
        pend_o[c] = pltpu.async_copy(
            wv.at[pl.ds(0, sz)], out_hbm.at[pl.ds(base + st, sz)], semo[ws])
    for c in sorted(pend_o):
        pend_o.pop(c).wait()


def kernel(input_ids, position_ids, token_type_ids, word_embeddings,
           position_embeddings, token_type_embeddings):
    B, S = input_ids.shape
    idw = input_ids.reshape(N_TOK).astype(jnp.int32)
    idp = position_ids.reshape(N_TOK).astype(jnp.int32)
    idt = token_type_ids.reshape(N_TOK).astype(jnp.int32)
    out = _emb_kernel(idw, idp, idt, word_embeddings.astype(jnp.float32),
                      position_embeddings.astype(jnp.float32),
                      token_type_embeddings.astype(jnp.float32))
    return out.reshape(B, S, HIDDEN)


# speedup vs baseline: 1.0605x; 1.0605x over previous
import functools

import jax
import jax.numpy as jnp
from jax import lax
from jax.experimental import pallas as pl
from jax.experimental.pallas import tpu as pltpu
from jax.experimental.pallas import tpu_sc as plsc

HIDDEN = 768
N_TOK = 8192
NC, NS, L = 2, 16, 16
NW = NC * NS
TOK_W = N_TOK // NW
TMAX = 32
HB = HIDDEN // L
U = 8
NWB = 3
NPB = 2

_SIZES = [8, 8, 16] + [32] * 7
assert sum(_SIZES) == TOK_W
_STARTS = [sum(_SIZES[:i]) for i in range(len(_SIZES))]
CHUNKS = list(zip(_STARTS, _SIZES))

_mesh = plsc.VectorSubcoreMesh(core_axis_name="c", subcore_axis_name="s")


@functools.partial(
    pl.kernel,
    mesh=_mesh,
    compiler_params=pltpu.CompilerParams(needs_layout_passes=False),
    out_type=jax.ShapeDtypeStruct((N_TOK, HIDDEN), jnp.float32),
    scratch_types=[
        pltpu.VMEM((TOK_W,), jnp.int32),
        pltpu.VMEM((TOK_W,), jnp.int32),
        pltpu.VMEM((TOK_W,), jnp.int32),
        pltpu.VMEM((NWB, TMAX, HIDDEN), jnp.float32),
        pltpu.VMEM((NPB, TMAX, HIDDEN), jnp.float32),
        pltpu.VMEM((2, HIDDEN), jnp.float32),
        pltpu.VMEM((TMAX, L), jnp.int32),
        pltpu.SemaphoreType.DMA,
        pltpu.SemaphoreType.DMA,
        pltpu.SemaphoreType.DMA,
        pltpu.SemaphoreType.DMA,
        pltpu.SemaphoreType.DMA,
        pltpu.SemaphoreType.DMA,
        pltpu.SemaphoreType.DMA,
        pltpu.SemaphoreType.DMA,
    ],
)
def _emb_kernel(idw_hbm, idp_hbm, idt_hbm, wtab_hbm, ptab_hbm, ttab_hbm,
                out_hbm, idw_v, idp_v, idt_v, wbuf, pbuf, tbuf, tidb_v,
                semw0, semw1, semw2, semp0, semp1, semo0, semo1, semo2):
    semw = (semw0, semw1, semw2)
    semp = (semp0, semp1)
    semo = (semo0, semo1, semo2)
    wid = lax.axis_index("s") * NC + lax.axis_index("c")
    base = wid * TOK_W
    pltpu.sync_copy(idw_hbm.at[pl.ds(base, TOK_W)], idw_v)
    pltpu.sync_copy(idp_hbm.at[pl.ds(base, TOK_W)], idp_v)
    pltpu.sync_copy(idt_hbm.at[pl.ds(base, TOK_W)], idt_v)
    pltpu.sync_copy(ttab_hbm, tbuf)

    def gathers(c):
        st, sz = CHUNKS[c]
        ws, ps = c % NWB, c % NPB
        gw = pltpu.async_copy(wtab_hbm.at[idw_v.at[pl.ds(st, sz)]],
                              wbuf.at[ws].at[pl.ds(0, sz)], semw[ws])
        gp = pltpu.async_copy(ptab_hbm.at[idp_v.at[pl.ds(st, sz)]],
                              pbuf.at[ps].at[pl.ds(0, sz)], semp[ps])
        return gw, gp

    pend_g = {0: gathers(0)}
    pend_o = {}
    for c in range(len(CHUNKS)):
        st, sz = CHUNKS[c]
        ws = c % NWB
        if c + 1 < len(CHUNKS):
            if c - 2 >= 0:
                pend_o.pop(c - 2).wait()
            pend_g[c + 1] = gathers(c + 1)
        gw, gp = pend_g.pop(c)
        gw.wait()
        gp.wait()

        wv = wbuf.at[ws]
        pv = pbuf.at[c % NPB]

        @plsc.parallel_loop(0, sz, 1, unroll=4)
        def pre_body(t):
            tidb_v[t] = plsc.load_gather(
                idt_v, [jnp.full((L,), st + t, jnp.int32)])

        def h_body(h, _):
            h16 = h * L
            a0 = tbuf[0, pl.ds(h16, L)]
            a1 = tbuf[1, pl.ds(h16, L)]

            @plsc.parallel_loop(0, sz, 1, unroll=min(U, sz))
            def t_loop(t):
                w = wv[t, pl.ds(h16, L)]
                p = pv[t, pl.ds(h16, L)]
                ti = tidb_v[t]
                wv[t, pl.ds(h16, L)] = w + p + jnp.where(ti != 0, a1, a0)

            return 0

        lax.fori_loop(0, HB, h_body, 0)
        pend_o[c] = pltpu.async_copy(
            wv.at[pl.ds(0, sz)], out_hbm.at[pl.ds(base + st, sz)], semo[ws])
    for c in sorted(pend_o):
        pend_o.pop(c).wait()


def kernel(input_ids, position_ids, token_type_ids, word_embeddings,
           position_embeddings, token_type_embeddings):
    B, S = input_ids.shape
    idw = input_ids.reshape(N_TOK).astype(jnp.int32)
    idp = position_ids.reshape(N_TOK).astype(jnp.int32)
    idt = token_type_ids.reshape(N_TOK).astype(jnp.int32)
    out = _emb_kernel(idw, idp, idt, word_embeddings.astype(jnp.float32),
                      position_embeddings.astype(jnp.float32),
                      token_type_embeddings.astype(jnp.float32))
    return out.reshape(B, S, HIDDEN)


# speedup vs baseline: 1.1116x; 1.0483x over previous
import functools

import jax
import jax.numpy as jnp
from jax import lax
from jax.experimental import pallas as pl
from jax.experimental.pallas import tpu as pltpu
from jax.experimental.pallas import tpu_sc as plsc

HIDDEN = 768
N_TOK = 8192
NC, NS, L = 2, 16, 16
NW = NC * NS
TOK_W = N_TOK // NW
TMAX = 32
HB = HIDDEN // L
U = 8
NWB = 3
NPB = 2

_SIZES = [8, 8, 16] + [32] * 7
assert sum(_SIZES) == TOK_W
_STARTS = [sum(_SIZES[:i]) for i in range(len(_SIZES))]
CHUNKS = list(zip(_STARTS, _SIZES))

_mesh = plsc.VectorSubcoreMesh(core_axis_name="c", subcore_axis_name="s")


@functools.partial(
    pl.kernel,
    mesh=_mesh,
    compiler_params=pltpu.CompilerParams(needs_layout_passes=False),
    out_type=jax.ShapeDtypeStruct((N_TOK, HIDDEN), jnp.float32),
    scratch_types=[
        pltpu.VMEM((TOK_W,), jnp.int32),
        pltpu.VMEM((TOK_W,), jnp.int32),
        pltpu.VMEM((TOK_W,), jnp.int32),
        pltpu.VMEM((NWB, TMAX, HIDDEN), jnp.float32),
        pltpu.VMEM((NPB, TMAX, HIDDEN), jnp.float32),
        pltpu.VMEM((2, HIDDEN), jnp.float32),
        pltpu.VMEM((TMAX, L), jnp.int32),
        pltpu.SemaphoreType.DMA,
        pltpu.SemaphoreType.DMA,
        pltpu.SemaphoreType.DMA,
        pltpu.SemaphoreType.DMA,
        pltpu.SemaphoreType.DMA,
        pltpu.SemaphoreType.DMA,
        pltpu.SemaphoreType.DMA,
        pltpu.SemaphoreType.DMA,
    ],
)
def _emb_kernel(idw_hbm, idp_hbm, idt_hbm, wtab_hbm, ptab_hbm, ttab_hbm,
                out_hbm, idw_v, idp_v, idt_v, wbuf, pbuf, tbuf, tidb_v,
                semw0, semw1, semw2, semp0, semp1, semo0, semo1, semo2):
    semw = (semw0, semw1, semw2)
    semp = (semp0, semp1)
    semo = (semo0, semo1, semo2)
    wid = lax.axis_index("s") * NC + lax.axis_index("c")
    base = wid * TOK_W
    h1 = pltpu.async_copy(idw_hbm.at[pl.ds(base, TOK_W)], idw_v, semo0)
    h2 = pltpu.async_copy(idp_hbm.at[pl.ds(base, TOK_W)], idp_v, semo1)
    h3 = pltpu.async_copy(idt_hbm.at[pl.ds(base, TOK_W)], idt_v, semo2)
    h4 = pltpu.async_copy(ttab_hbm, tbuf, semp0)
    h1.wait()
    h2.wait()

    def gathers(c):
        st, sz = CHUNKS[c]
        ws, ps = c % NWB, c % NPB
        gw = pltpu.async_copy(wtab_hbm.at[idw_v.at[pl.ds(st, sz)]],
                              wbuf.at[ws].at[pl.ds(0, sz)], semw[ws])
        gp = pltpu.async_copy(ptab_hbm.at[idp_v.at[pl.ds(st, sz)]],
                              pbuf.at[ps].at[pl.ds(0, sz)], semp[ps])
        return gw, gp

    pend_g = {0: gathers(0)}
    h3.wait()
    h4.wait()
    pend_o = {}
    for c in range(len(CHUNKS)):
        st, sz = CHUNKS[c]
        ws = c % NWB
        if c + 1 < len(CHUNKS):
            if c - 2 >= 0:
                pend_o.pop(c - 2).wait()
            pend_g[c + 1] = gathers(c + 1)
        gw, gp = pend_g.pop(c)
        gw.wait()
        gp.wait()

        wv = wbuf.at[ws]
        pv = pbuf.at[c % NPB]

        @plsc.parallel_loop(0, sz, 1, unroll=4)
        def pre_body(t):
            tidb_v[t] = plsc.load_gather(
                idt_v, [jnp.full((L,), st + t, jnp.int32)])

        def h_body(h, _):
            h16 = h * L
            a0 = tbuf[0, pl.ds(h16, L)]
            a1 = tbuf[1, pl.ds(h16, L)]

            @plsc.parallel_loop(0, sz, 1, unroll=min(U, sz))
            def t_loop(t):
                w = wv[t, pl.ds(h16, L)]
                p = pv[t, pl.ds(h16, L)]
                ti = tidb_v[t]
                wv[t, pl.ds(h16, L)] = w + p + jnp.where(ti != 0, a1, a0)

            return 0

        lax.fori_loop(0, HB, h_body, 0)
        pend_o[c] = pltpu.async_copy(
            wv.at[pl.ds(0, sz)], out_hbm.at[pl.ds(base + st, sz)], semo[ws])
    for c in sorted(pend_o):
        pend_o.pop(c).wait()


def kernel(input_ids, position_ids, token_type_ids, word_embeddings,
           position_embeddings, token_type_embeddings):
    B, S = input_ids.shape
    idw = input_ids.reshape(N_TOK).astype(jnp.int32)
    idp = position_ids.reshape(N_TOK).astype(jnp.int32)
    idt = token_type_ids.reshape(N_TOK).astype(jnp.int32)
    out = _emb_kernel(idw, idp, idt, word_embeddings.astype(jnp.float32),
                      position_embeddings.astype(jnp.float32),
                      token_type_embeddings.astype(jnp.float32))
    return out.reshape(B, S, HIDDEN)


# speedup vs baseline: 1.1132x; 1.0014x over previous
import functools

import jax
import jax.numpy as jnp
from jax import lax
from jax.experimental import pallas as pl
from jax.experimental.pallas import tpu as pltpu
from jax.experimental.pallas import tpu_sc as plsc

HIDDEN = 768
N_TOK = 8192
NC, NS, L = 2, 16, 16
NW = NC * NS
TOK_W = N_TOK // NW
TMAX = 32
HB = HIDDEN // L
U = 8
NWB = 3
NPB = 2

_SIZES = [8, 8, 16] + [32] * 7
assert sum(_SIZES) == TOK_W
_STARTS = [sum(_SIZES[:i]) for i in range(len(_SIZES))]
CHUNKS = list(zip(_STARTS, _SIZES))

_mesh = plsc.VectorSubcoreMesh(core_axis_name="c", subcore_axis_name="s")


@functools.partial(
    pl.kernel,
    mesh=_mesh,
    compiler_params=pltpu.CompilerParams(needs_layout_passes=False),
    out_type=jax.ShapeDtypeStruct((N_TOK, HIDDEN), jnp.float32),
    scratch_types=[
        pltpu.VMEM((TOK_W,), jnp.int32),
        pltpu.VMEM((TOK_W,), jnp.int32),
        pltpu.VMEM((TOK_W,), jnp.int32),
        pltpu.VMEM((NWB, TMAX, HIDDEN), jnp.float32),
        pltpu.VMEM((NPB, TMAX, HIDDEN), jnp.float32),
        pltpu.VMEM((2, HIDDEN), jnp.float32),
        pltpu.VMEM((TMAX, L), jnp.int32),
        pltpu.SemaphoreType.DMA,
        pltpu.SemaphoreType.DMA,
        pltpu.SemaphoreType.DMA,
        pltpu.SemaphoreType.DMA,
        pltpu.SemaphoreType.DMA,
        pltpu.SemaphoreType.DMA,
        pltpu.SemaphoreType.DMA,
        pltpu.SemaphoreType.DMA,
    ],
)
def _emb_kernel(idw_hbm, idp_hbm, idt_hbm, wtab_hbm, ptab_hbm, ttab_hbm,
                out_hbm, idw_v, idp_v, idt_v, wbuf, pbuf, tbuf, tidb_v,
                semw0, semw1, semw2, semp0, semp1, semo0, semo1, semo2):
    semw = (semw0, semw1, semw2)
    semp = (semp0, semp1)
    semo = (semo0, semo1, semo2)
    wid = lax.axis_index("s") * NC + lax.axis_index("c")
    base = wid * TOK_W
    h1 = pltpu.async_copy(idw_hbm.at[pl.ds(base, TOK_W)], idw_v, semo0)
    h2 = pltpu.async_copy(idp_hbm.at[pl.ds(base, TOK_W)], idp_v, semo1)
    h3 = pltpu.async_copy(idt_hbm.at[pl.ds(base, TOK_W)], idt_v, semo2)
    h4 = pltpu.async_copy(ttab_hbm, tbuf, semp0)
    h1.wait()
    h2.wait()

    def gathers(c):
        st, sz = CHUNKS[c]
        ws, ps = c % NWB, c % NPB
        gw = pltpu.async_copy(wtab_hbm.at[idw_v.at[pl.ds(st, sz)]],
                              wbuf.at[ws].at[pl.ds(0, sz)], semw[ws])
        gp = pltpu.async_copy(ptab_hbm.at[idp_v.at[pl.ds(st, sz)]],
                              pbuf.at[ps].at[pl.ds(0, sz)], semp[ps])
        return gw, gp

    pend_g = {0: gathers(0)}
    h3.wait()
    h4.wait()
    pend_o = {}
    for c in range(len(CHUNKS)):
        st, sz = CHUNKS[c]
        ws = c % NWB
        if c + 1 < len(CHUNKS):
            if c - 2 >= 0:
                pend_o.pop(c - 2).wait()
            pend_g[c + 1] = gathers(c + 1)
        @plsc.parallel_loop(0, sz, 1, unroll=4)
        def pre_body(t):
            tidb_v[t] = plsc.load_gather(
                idt_v, [jnp.full((L,), st + t, jnp.int32)])

        gw, gp = pend_g.pop(c)
        gw.wait()
        gp.wait()

        wv = wbuf.at[ws]
        pv = pbuf.at[c % NPB]

        def h_body(h, _):
            h16 = h * L
            a0 = tbuf[0, pl.ds(h16, L)]
            a1 = tbuf[1, pl.ds(h16, L)]

            @plsc.parallel_loop(0, sz, 1, unroll=min(U, sz))
            def t_loop(t):
                w = wv[t, pl.ds(h16, L)]
                p = pv[t, pl.ds(h16, L)]
                ti = tidb_v[t]
                wv[t, pl.ds(h16, L)] = w + p + jnp.where(ti != 0, a1, a0)

            return 0

        lax.fori_loop(0, HB, h_body, 0)
        pend_o[c] = pltpu.async_copy(
            wv.at[pl.ds(0, sz)], out_hbm.at[pl.ds(base + st, sz)], semo[ws])
    for c in sorted(pend_o):
        pend_o.pop(c).wait()


def kernel(input_ids, position_ids, token_type_ids, word_embeddings,
           position_embeddings, token_type_embeddings):
    B, S = input_ids.shape
    idw = input_ids.reshape(N_TOK).astype(jnp.int32)
    idp = position_ids.reshape(N_TOK).astype(jnp.int32)
    idt = token_type_ids.reshape(N_TOK).astype(jnp.int32)
    out = _emb_kernel(idw, idp, idt, word_embeddings.astype(jnp.float32),
                      position_embeddings.astype(jnp.float32),
                      token_type_embeddings.astype(jnp.float32))
    return out.reshape(B, S, HIDDEN)
